# baseline (speedup 1.0000x reference)
import functools

import jax
import jax.numpy as jnp
from jax import lax
from jax.experimental import pallas as pl
from jax.experimental.pallas import tpu as pltpu
from jax.experimental.pallas import tpu_sc as plsc

_B = 16384
_F = 26
_D = 16
_FIELD = 38462

_NC = 2
_NS = 16
_NW = _NC * _NS

_TOTAL = _B * _F
_GW = 128
_ROWS = _TOTAL // _GW
_RPW = _ROWS // _NW
_K = 8
_NG = _RPW // _K


def _body(table_hbm, xr_hbm, out_hbm, idx_v, rows_v, sem):
    wid = lax.axis_index("s") * _NC + lax.axis_index("c")
    row0 = wid * _RPW

    pltpu.sync_copy(xr_hbm.at[pl.ds(row0, _RPW)], idx_v)

    lanes = lax.iota(jnp.int32, 16)

    def add_off(r, carry):
        for c in range(_GW // 16):
            pos0 = (row0 + r) * _GW + c * 16
            fld = (pos0 + lanes) % _F
            sl = pl.ds(c * 16, 16)
            idx_v[r, sl] = idx_v[r, sl] + fld * _FIELD
        return carry

    lax.fori_loop(0, _RPW, add_off, 0)

    def gather_group(g, carry):
        base_r = g * _K
        cps = []
        for j in range(_K):
            cps.append(
                pltpu.async_copy(
                    table_hbm.at[idx_v.at[base_r + j]], rows_v.at[j], sem
                )
            )
        for cp in cps:
            cp.wait()
        pltpu.sync_copy(rows_v, out_hbm.at[pl.ds(row0 + base_r, _K)])
        return carry

    lax.fori_loop(0, _NG, gather_group, 0)


@jax.jit
def _embed(x, table):
    xr = x.reshape(_ROWS, _GW)
    mesh = plsc.VectorSubcoreMesh(core_axis_name="c", subcore_axis_name="s")
    run = functools.partial(
        pl.kernel,
        mesh=mesh,
        out_type=jax.ShapeDtypeStruct((_ROWS, _GW, _D), jnp.float32),
        scratch_types=[
            pltpu.VMEM((_RPW, _GW), jnp.int32),
            pltpu.VMEM((_K, _GW, _D), jnp.float32),
            pltpu.SemaphoreType.DMA,
        ],
        compiler_params=pltpu.CompilerParams(use_tc_tiling_on_sc=False),
    )(_body)
    out = run(table, xr)
    return out.reshape(_B, _F, _D)


def kernel(x, table):
    return _embed(x, table)


# speedup vs baseline: 1.0120x; 1.0120x over previous
import functools

import jax
import jax.numpy as jnp
from jax import lax
from jax.experimental import pallas as pl
from jax.experimental.pallas import tpu as pltpu
from jax.experimental.pallas import tpu_sc as plsc

_B = 16384
_F = 26
_D = 16
_FIELD = 38462

_NC = 2
_NS = 16
_NW = _NC * _NS

_TOTAL = _B * _F
_GW = 128
_ROWS = _TOTAL // _GW
_RPW = _ROWS // _NW
_K = 13
_NG = _RPW // _K


def _body(table_hbm, xr_hbm, out_hbm, idx_v, rows0, rows1, gs0, gs1, ss0, ss1):
    wid = lax.axis_index("s") * _NC + lax.axis_index("c")
    row0 = wid * _RPW

    pltpu.sync_copy(xr_hbm.at[pl.ds(row0, _RPW)], idx_v)

    lanes = lax.iota(jnp.int32, 16)

    def add_off(r, carry):
        for c in range(_GW // 16):
            pos0 = (row0 + r) * _GW + c * 16
            fld = (pos0 + lanes) % _F
            sl = pl.ds(c * 16, 16)
            idx_v[r, sl] = idx_v[r, sl] + fld * _FIELD
        return carry

    lax.fori_loop(0, _RPW, add_off, 0)

    def fire(g, rows, gsem):
        base = g * _K
        for j in range(_K):
            pltpu.async_copy(table_hbm.at[idx_v.at[base + j]], rows.at[j], gsem)

    def drain(rows, sem):
        pltpu.make_async_copy(out_hbm.at[pl.ds(row0, _K)], rows, sem).wait()

    def store_fire(g, rows, ssem):
        pltpu.async_copy(rows, out_hbm.at[pl.ds(row0 + g * _K, _K)], ssem)

    fire(0, rows0, gs0)

    def step(k, mr, mg, ms, orr, og, osem):
        @pl.when(k < _NG - 1)
        def _():
            @pl.when(k >= 1)
            def _():
                drain(orr, osem)
            fire(k + 1, orr, og)

        drain(mr, mg)
        store_fire(k, mr, ms)

    def pair(p, carry):
        step(2 * p, rows0, gs0, ss0, rows1, gs1, ss1)
        step(2 * p + 1, rows1, gs1, ss1, rows0, gs0, ss0)
        return carry

    lax.fori_loop(0, _NG // 2, pair, 0)

    drain(rows0, ss0)
    drain(rows1, ss1)


@jax.jit
def _embed(x, table):
    xr = x.reshape(_ROWS, _GW)
    mesh = plsc.VectorSubcoreMesh(core_axis_name="c", subcore_axis_name="s")
    run = functools.partial(
        pl.kernel,
        mesh=mesh,
        out_type=jax.ShapeDtypeStruct((_ROWS, _GW, _D), jnp.float32),
        scratch_types=[
            pltpu.VMEM((_RPW, _GW), jnp.int32),
            pltpu.VMEM((_K, _GW, _D), jnp.float32),
            pltpu.VMEM((_K, _GW, _D), jnp.float32),
            pltpu.SemaphoreType.DMA,
            pltpu.SemaphoreType.DMA,
            pltpu.SemaphoreType.DMA,
            pltpu.SemaphoreType.DMA,
        ],
        compiler_params=pltpu.CompilerParams(use_tc_tiling_on_sc=False),
    )(_body)
    out = run(table, xr)
    return out.reshape(_B, _F, _D)


def kernel(x, table):
    return _embed(x, table)


# speedup vs baseline: 5.2861x; 5.2232x over previous
import functools

import jax
import jax.numpy as jnp
from jax import lax
from jax.experimental import pallas as pl
from jax.experimental.pallas import tpu as pltpu
from jax.experimental.pallas import tpu_sc as plsc

_B = 16384
_F = 26
_D = 16
_FIELD = 38462
_NROWS = _F * _FIELD
_TAIL0 = (_NROWS // 128) * 128
_NTAIL = _NROWS - _TAIL0
_W = 38656
_NV = _B // 16


def _win(f):
    lo = f * _FIELD
    c0 = (lo // 128) * 128
    if c0 + _W <= _NROWS:
        return c0, _W, lo - c0
    return c0, _TAIL0 - c0, lo - c0


def _emit_half(fields, t2, xs, out3, tbl_v, idx0, out_v, tail_v,
               spt0, spo0, sem_tbl, sem_idx, ss0, s):
    nf = len(fields)
    is0 = s == 0

    def store_cp(k):
        return pltpu.make_async_copy(spo0, out3.at[fields[k]], ss0)

    def tbl_cp(k):
        c0, w, _ = _win(fields[k])
        return pltpu.make_async_copy(
            t2.at[:, pl.ds(c0, w)], spt0.at[:, pl.ds(0, w)], sem_tbl
        )

    def idx_cp(k):
        return pltpu.make_async_copy(xs.at[fields[k], :], idx0, sem_idx)

    @pl.when(is0)
    def _():
        tbl_cp(0).start()

    idx_cp(0).start()

    for k in range(nf):
        p = k % 2
        f = fields[k]
        _, _, shift = _win(f)

        @pl.when(is0)
        def _():
            tbl_cp(k).wait()

        plsc.subcore_barrier()

        idx_cp(k).wait()

        pltpu.sync_copy(spt0.at[s], tbl_v)
        plsc.subcore_barrier()

        if k + 1 < nf:
            @pl.when(is0)
            def _():
                tbl_cp(k + 1).start()

        iv = idx0
        if f == _F - 1:
            lim = _TAIL0 - f * _FIELD
            dsplat = jnp.zeros((16,), jnp.int32) + s

            def gstep(i, c):
                for u in range(8):
                    sl = pl.ds((i * 8 + u) * 16, 16)
                    xv = iv[sl]
                    va = plsc.load_gather(tbl_v, [xv + shift])
                    ti = lax.max(xv - lim, 0)
                    vb = plsc.load_gather(tail_v, [dsplat, ti])
                    out_v[sl] = jnp.where(xv >= lim, vb, va)
                return c
        else:
            def gstep(i, c):
                for u in range(8):
                    sl = pl.ds((i * 8 + u) * 16, 16)
                    out_v[sl] = plsc.load_gather(tbl_v, [iv[sl] + shift])
                return c

        lax.fori_loop(0, _NV // 8, gstep, 0)

        if k + 1 < nf:
            idx_cp(k + 1).start()

        if k >= 1:
            @pl.when(is0)
            def _():
                store_cp(k - 1).wait()

            plsc.subcore_barrier()

        pltpu.sync_copy(out_v, spo0.at[s])
        plsc.subcore_barrier()

        @pl.when(is0)
        def _():
            store_cp(k).start()

    @pl.when(is0)
    def _():
        store_cp(nf - 1).wait()


def _body(t2, xs, tail, out3, tbl_v, idx0, out_v, tail_v,
          spt0, spo0, sem_tbl, sem_idx, ss0):
    c = lax.axis_index("c")
    s = lax.axis_index("s")

    pltpu.sync_copy(tail, tail_v)

    args = (t2, xs, out3, tbl_v, idx0, out_v, tail_v,
            spt0, spo0, sem_tbl, sem_idx, ss0, s)

    @pl.when(c == 0)
    def _():
        _emit_half(list(range(13)), *args)

    @pl.when(c == 1)
    def _():
        _emit_half(list(range(13, 26)), *args)


@jax.jit
def _embed(x, table):
    t2 = table.T
    xs = x.T
    tail = jnp.pad(t2[:, _TAIL0:], ((0, 0), (0, 128 - _NTAIL)))
    mesh = plsc.VectorSubcoreMesh(core_axis_name="c", subcore_axis_name="s")
    run = functools.partial(
        pl.kernel,
        mesh=mesh,
        out_type=jax.ShapeDtypeStruct((_F, _D, _B), jnp.float32),
        scratch_types=[
            pltpu.VMEM((_W,), jnp.float32),
            pltpu.VMEM((_B,), jnp.int32),
            pltpu.VMEM((_B,), jnp.float32),
            pltpu.VMEM((_D, 128), jnp.float32),
            pltpu.VMEM_SHARED((_D, _W), jnp.float32),
            pltpu.VMEM_SHARED((_D, _B), jnp.float32),
            pltpu.SemaphoreType.DMA,
            pltpu.SemaphoreType.DMA,
            pltpu.SemaphoreType.DMA,
        ],
        compiler_params=pltpu.CompilerParams(needs_layout_passes=False),
    )(_body)
    out3 = run(t2, xs, tail)
    return out3.transpose(2, 0, 1)


def kernel(x, table):
    return _embed(x, table)


# speedup vs baseline: 5.9076x; 1.1176x over previous
import functools

import jax
import jax.numpy as jnp
from jax import lax
from jax.experimental import pallas as pl
from jax.experimental.pallas import tpu as pltpu
from jax.experimental.pallas import tpu_sc as plsc

_B = 16384
_F = 26
_D = 16
_FIELD = 38462
_NROWS = _F * _FIELD
_TAIL0 = (_NROWS // 128) * 128
_NTAIL = _NROWS - _TAIL0
_W = 38656
_NV = _B // 16


def _win(f):
    lo = f * _FIELD
    c0 = (lo // 128) * 128
    if c0 + _W <= _NROWS:
        return c0, _W, lo - c0
    return c0, _TAIL0 - c0, lo - c0


def _emit_half(fields, t2, xs, out3, tbl_v, idx0, out_v, tail_v,
               spt0, spo0, sem_tbl, sem_idx, sem_pull, ss0, s):
    nf = len(fields)
    is0 = s == 0

    def tbl_cp(k):
        c0, w, _ = _win(fields[k])
        return pltpu.make_async_copy(
            t2.at[:, pl.ds(c0, w)], spt0.at[:, pl.ds(0, w)], sem_tbl
        )

    def idx_cp(k):
        return pltpu.make_async_copy(xs.at[fields[k], :], idx0, sem_idx)

    def store_cp(k):
        return pltpu.make_async_copy(spo0, out3.at[fields[k]], ss0)

    def pull_cp():
        return pltpu.make_async_copy(spt0.at[s], tbl_v, sem_pull)

    @pl.when(is0)
    def _():
        tbl_cp(0).start()

    idx_cp(0).start()

    for k in range(nf):
        f = fields[k]
        _, _, shift = _win(f)

        @pl.when(is0)
        def _():
            tbl_cp(k).wait()

        plsc.subcore_barrier()

        pull_cp().start()

        if k >= 1:
            if k >= 2:
                @pl.when(is0)
                def _():
                    store_cp(k - 2).wait()

                plsc.subcore_barrier()

            pltpu.sync_copy(out_v, spo0.at[s])
            plsc.subcore_barrier()

            @pl.when(is0)
            def _():
                store_cp(k - 1).start()

        idx_cp(k).wait()
        pull_cp().wait()
        plsc.subcore_barrier()

        if k + 1 < nf:
            @pl.when(is0)
            def _():
                tbl_cp(k + 1).start()

        iv = idx0
        if f == _F - 1:
            lim = _TAIL0 - f * _FIELD
            dsplat = jnp.zeros((16,), jnp.int32) + s

            def gstep(i, cc):
                for u in range(8):
                    sl = pl.ds((i * 8 + u) * 16, 16)
                    xv = iv[sl]
                    va = plsc.load_gather(tbl_v, [xv + shift])
                    ti = lax.max(xv - lim, 0)
                    vb = plsc.load_gather(tail_v, [dsplat, ti])
                    out_v[sl] = jnp.where(xv >= lim, vb, va)
                return cc

            lax.fori_loop(0, _NV // 8, gstep, 0)
        else:
            def gstep(i, cc):
                for u in range(8):
                    sl = pl.ds((i * 8 + u) * 16, 16)
                    out_v[sl] = plsc.load_gather(tbl_v, [iv[sl] + shift])
                return cc

            lax.fori_loop(0, _NV // 8, gstep, 0)

        if k + 1 < nf:
            idx_cp(k + 1).start()

    @pl.when(is0)
    def _():
        store_cp(nf - 2).wait()

    plsc.subcore_barrier()
    pltpu.sync_copy(out_v, spo0.at[s])
    plsc.subcore_barrier()

    @pl.when(is0)
    def _():
        store_cp(nf - 1).start()
        store_cp(nf - 1).wait()


def _body(t2, xs, tail, out3, tbl_v, idx0, out_v, tail_v,
          spt0, spo0, sem_tbl, sem_idx, sem_pull, ss0):
    c = lax.axis_index("c")
    s = lax.axis_index("s")

    pltpu.sync_copy(tail, tail_v)

    args = (t2, xs, out3, tbl_v, idx0, out_v, tail_v,
            spt0, spo0, sem_tbl, sem_idx, sem_pull, ss0, s)

    @pl.when(c == 0)
    def _():
        _emit_half(list(range(13)), *args)

    @pl.when(c == 1)
    def _():
        _emit_half(list(range(13, 26)), *args)


@jax.jit
def _embed(x, table):
    t2 = table.T
    xs = x.T
    tail = jnp.pad(t2[:, _TAIL0:], ((0, 0), (0, 128 - _NTAIL)))
    mesh = plsc.VectorSubcoreMesh(core_axis_name="c", subcore_axis_name="s")
    run = functools.partial(
        pl.kernel,
        mesh=mesh,
        out_type=jax.ShapeDtypeStruct((_F, _D, _B), jnp.float32),
        scratch_types=[
            pltpu.VMEM((_W,), jnp.float32),
            pltpu.VMEM((_B,), jnp.int32),
            pltpu.VMEM((_B,), jnp.float32),
            pltpu.VMEM((_D, 128), jnp.float32),
            pltpu.VMEM_SHARED((_D, _W), jnp.float32),
            pltpu.VMEM_SHARED((_D, _B), jnp.float32),
            pltpu.SemaphoreType.DMA,
            pltpu.SemaphoreType.DMA,
            pltpu.SemaphoreType.DMA,
            pltpu.SemaphoreType.DMA,
        ],
        compiler_params=pltpu.CompilerParams(needs_layout_passes=False),
    )(_body)
    out3 = run(t2, xs, tail)
    return out3.transpose(2, 0, 1)


def kernel(x, table):
    return _embed(x, table)
